# baseline (speedup 1.0000x reference)
import functools

import jax
import jax.numpy as jnp
from jax import lax
from jax.experimental import pallas as pl
from jax.experimental.pallas import tpu as pltpu
from jax.experimental.pallas import tpu_sc as plsc

_NUM_SLOTS = 8192
_NUM_UPD = 4096
_D = 128

_NC = 2
_NS = 16
_NW = _NC * _NS

_SLOTS_W = _NUM_SLOTS // _NW

_R = 6
_LOOK = _R // 2
_L = 16
_ROWS_P = _R * ((_SLOTS_W + _LOOK + _R - 1) // _R)
_JW_SZ = _ROWS_P + _LOOK + _L


def _sload(ref, i):
    return ref[pl.ds(i, _L)][0]


def _sstore(ref, i, val, lane0):
    plsc.store_scatter(
        ref,
        [jnp.full((_L,), i, jnp.int32)],
        jnp.full((_L,), val, ref.dtype),
        mask=lane0,
    )


_sc_mesh = plsc.VectorSubcoreMesh(
    core_axis_name="c", subcore_axis_name="s", num_cores=_NC, num_subcores=_NS
)


@functools.partial(
    pl.kernel,
    out_type=jax.ShapeDtypeStruct((_NUM_SLOTS, _D, _D), jnp.float32),
    mesh=_sc_mesh,
    compiler_params=pltpu.CompilerParams(needs_layout_passes=False),
    scratch_types=[
        pltpu.VMEM((_NUM_UPD + _L,), jnp.int32),
        pltpu.VMEM((_JW_SZ,), jnp.int32),
        pltpu.VMEM((_R, _D, _D), jnp.float32),
        pltpu.SemaphoreType.DMA((_R,)),
        pltpu.SemaphoreType.DMA((_R,)),
    ],
)
def _sc_update(idx_hbm, arg0_hbm, arg3_hbm, out_hbm,
               idx_v, jwin, bufs, in_sems, out_sems):
    wid = lax.axis_index("s") * _NC + lax.axis_index("c")
    ii0 = lax.iota(jnp.int32, _L)
    lane0 = ii0 == 0

    pltpu.sync_copy(idx_hbm, idx_v.at[pl.ds(0, _NUM_UPD)])

    for b in range(_JW_SZ // _L):
        jwin[pl.ds(b * _L, _L)] = jnp.full((_L,), -1, jnp.int32)

    def sel_body(j, _):
        d = _sload(idx_v, j)
        mine = (d & (_NW - 1)) == wid

        @pl.when(mine)
        def _():
            _sstore(jwin, lax.shift_right_logical(d, 5), j, lane0)

        return 0

    lax.fori_loop(0, _NUM_UPD, sel_body, 0)

    jlast = _sload(jwin, _SLOTS_W - 1)

    def pad_body(p, _):
        _sstore(jwin, p, jlast, lane0)
        return 0

    lax.fori_loop(_SLOTS_W, _ROWS_P, pad_body, 0)

    zero16 = jnp.zeros((_L,), jnp.float32)

    def slot_of(p):
        return jnp.minimum(p, _SLOTS_W - 1) * _NW + wid

    def fire_in(p, b):
        j = _sload(jwin, p)

        @pl.when(j >= 0)
        def _():
            pltpu.async_copy(arg3_hbm.at[j], bufs.at[b], in_sems.at[b])

        @pl.when(j < 0)
        def _():
            pltpu.async_copy(
                arg0_hbm.at[slot_of(p)], bufs.at[b], in_sems.at[b]
            )

    def drain_in(b):
        pltpu.make_async_copy(
            arg0_hbm.at[0], bufs.at[b], in_sems.at[b]
        ).wait()

    def fire_out(p, b):
        pltpu.async_copy(bufs.at[b], out_hbm.at[slot_of(p)], out_sems.at[b])

    def drain_out(b):
        pltpu.make_async_copy(
            bufs.at[b], out_hbm.at[0], out_sems.at[b]
        ).wait()

    def zero_diag(b):
        for k in range(_D // _L):
            ii = ii0 + (_L * k)
            plsc.store_scatter(bufs.at[b], [ii, ii], zero16)

    for b in range(_LOOK):
        fire_in(b, b)

    def ring_body(i, _):
        r0 = _R * i
        for t in range(_R):
            r = r0 + t
            drain_in(t)
            zero_diag(t)
            fire_out(r, t)
            to = (t + _LOOK) % _R
            if t >= _LOOK:
                drain_out(to)
            else:

                @pl.when(r >= _LOOK)
                def _():
                    drain_out(to)

            fire_in(r + _LOOK, to)
        return 0

    lax.fori_loop(0, _ROWS_P // _R, ring_body, 0)

    for t in range(_LOOK):
        drain_out((_ROWS_P - _LOOK + t) % _R)
        drain_in((_ROWS_P + t) % _R)


@jax.jit
def kernel(arg0_1, arg1_1, arg2_1, arg3_1):
    del arg2_1
    idx = arg1_1.astype(jnp.int32)
    return _sc_update(idx, arg0_1, arg3_1)


# speedup vs baseline: 1.1613x; 1.1613x over previous
import functools

import jax
import jax.numpy as jnp
from jax import lax
from jax.experimental import pallas as pl
from jax.experimental.pallas import tpu as pltpu
from jax.experimental.pallas import tpu_sc as plsc

_NUM_SLOTS = 8192
_NUM_UPD = 4096
_D = 128

_NC = 2
_NS = 16
_NW = _NC * _NS

_SLOTS_W = _NUM_SLOTS // _NW

_R = 6
_LOOK = _R // 2
_L = 16
_ROWS_P = _R * ((_SLOTS_W + _LOOK + _R - 1) // _R)
_JW_SZ = _ROWS_P + _LOOK + _L


def _sload(ref, i):
    return ref[pl.ds(i, _L)][0]


def _sstore(ref, i, val, lane0):
    plsc.store_scatter(
        ref,
        [jnp.full((_L,), i, jnp.int32)],
        jnp.full((_L,), val, ref.dtype),
        mask=lane0,
    )


_sc_mesh = plsc.VectorSubcoreMesh(
    core_axis_name="c", subcore_axis_name="s", num_cores=_NC, num_subcores=_NS
)


@functools.partial(
    pl.kernel,
    out_type=jax.ShapeDtypeStruct((_NUM_SLOTS, _D, _D), jnp.float32),
    mesh=_sc_mesh,
    compiler_params=pltpu.CompilerParams(needs_layout_passes=False),
    scratch_types=[
        pltpu.VMEM((_NUM_UPD + _L,), jnp.int32),
        pltpu.VMEM((_JW_SZ,), jnp.int32),
        pltpu.VMEM((_R, _D, _D), jnp.float32),
        pltpu.SemaphoreType.DMA((_R,)),
        pltpu.SemaphoreType.DMA((_R,)),
    ],
)
def _sc_update(idx_hbm, arg0_hbm, arg3_hbm, out_hbm,
               idx_v, jwin, bufs, in_sems, out_sems):
    wid = lax.axis_index("s") * _NC + lax.axis_index("c")
    ii0 = lax.iota(jnp.int32, _L)
    lane0 = ii0 == 0

    pltpu.sync_copy(idx_hbm, idx_v.at[pl.ds(0, _NUM_UPD)])

    for b in range(_JW_SZ // _L):
        jwin[pl.ds(b * _L, _L)] = jnp.full((_L,), -1, jnp.int32)

    def sel_body(t, _):
        v = idx_v[pl.ds(t * _L, _L)]
        mine = (v & (_NW - 1)) == wid

        @pl.when(jnp.any(mine))
        def _():
            slots = lax.shift_right_logical(v, 5)
            jv = (t * _L) + ii0
            plsc.store_scatter(jwin, [slots], jv, mask=mine)
            g0 = plsc.load_gather(jwin, [slots], mask=mine)

            def fix_cond(g):
                return jnp.any(mine & (jv > g))

            def fix_body(g):
                plsc.store_scatter(jwin, [slots], jv, mask=mine & (jv > g))
                return plsc.load_gather(jwin, [slots], mask=mine)

            lax.while_loop(fix_cond, fix_body, g0)

        return 0

    lax.fori_loop(0, _NUM_UPD // _L, sel_body, 0)

    jlast = _sload(jwin, _SLOTS_W - 1)

    def pad_body(p, _):
        _sstore(jwin, p, jlast, lane0)
        return 0

    lax.fori_loop(_SLOTS_W, _ROWS_P, pad_body, 0)

    zero16 = jnp.zeros((_L,), jnp.float32)

    def slot_of(p):
        return jnp.minimum(p, _SLOTS_W - 1) * _NW + wid

    def fire_in(p, b):
        j = _sload(jwin, p)

        @pl.when(j >= 0)
        def _():
            pltpu.async_copy(arg3_hbm.at[j], bufs.at[b], in_sems.at[b])

        @pl.when(j < 0)
        def _():
            pltpu.async_copy(
                arg0_hbm.at[slot_of(p)], bufs.at[b], in_sems.at[b]
            )

    def drain_in(b):
        pltpu.make_async_copy(
            arg0_hbm.at[0], bufs.at[b], in_sems.at[b]
        ).wait()

    def fire_out(p, b):
        pltpu.async_copy(bufs.at[b], out_hbm.at[slot_of(p)], out_sems.at[b])

    def drain_out(b):
        pltpu.make_async_copy(
            bufs.at[b], out_hbm.at[0], out_sems.at[b]
        ).wait()

    def zero_diag(b):
        for k in range(_D // _L):
            ii = ii0 + (_L * k)
            plsc.store_scatter(bufs.at[b], [ii, ii], zero16)

    for b in range(_LOOK):
        fire_in(b, b)

    def ring_body(i, _):
        r0 = _R * i
        for t in range(_R):
            r = r0 + t
            drain_in(t)
            zero_diag(t)
            fire_out(r, t)
            to = (t + _LOOK) % _R
            if t >= _LOOK:
                drain_out(to)
            else:

                @pl.when(r >= _LOOK)
                def _():
                    drain_out(to)

            fire_in(r + _LOOK, to)
        return 0

    lax.fori_loop(0, _ROWS_P // _R, ring_body, 0)

    for t in range(_LOOK):
        drain_out((_ROWS_P - _LOOK + t) % _R)
        drain_in((_ROWS_P + t) % _R)


@jax.jit
def kernel(arg0_1, arg1_1, arg2_1, arg3_1):
    del arg2_1
    idx = arg1_1.astype(jnp.int32)
    return _sc_update(idx, arg0_1, arg3_1)


# speedup vs baseline: 1.1816x; 1.0175x over previous
import functools

import jax
import jax.numpy as jnp
from jax import lax
from jax.experimental import pallas as pl
from jax.experimental.pallas import tpu as pltpu
from jax.experimental.pallas import tpu_sc as plsc

_NUM_SLOTS = 8192
_NUM_UPD = 4096
_D = 128

_NC = 2
_NS = 16
_NW = _NC * _NS

_SLOTS_W = _NUM_SLOTS // _NW

_R = 7
_LOOK = 3
_LAG = _R - _LOOK
_L = 16
_ROWS_P = _R * ((_SLOTS_W + _LOOK + _R - 1) // _R)
_JW_SZ = _ROWS_P + _LOOK + _L


def _sload(ref, i):
    return ref[pl.ds(i, _L)][0]


def _sstore(ref, i, val, lane0):
    plsc.store_scatter(
        ref,
        [jnp.full((_L,), i, jnp.int32)],
        jnp.full((_L,), val, ref.dtype),
        mask=lane0,
    )


_sc_mesh = plsc.VectorSubcoreMesh(
    core_axis_name="c", subcore_axis_name="s", num_cores=_NC, num_subcores=_NS
)


@functools.partial(
    pl.kernel,
    out_type=jax.ShapeDtypeStruct((_NUM_SLOTS, _D, _D), jnp.float32),
    mesh=_sc_mesh,
    compiler_params=pltpu.CompilerParams(needs_layout_passes=False),
    scratch_types=[
        pltpu.VMEM((_NUM_UPD + _L,), jnp.int32),
        pltpu.VMEM((_JW_SZ,), jnp.int32),
        pltpu.VMEM((_R, _D, _D), jnp.float32),
        pltpu.SemaphoreType.DMA((_R,)),
        pltpu.SemaphoreType.DMA((_R,)),
    ],
)
def _sc_update(idx_hbm, arg0_hbm, arg3_hbm, out_hbm,
               idx_v, jwin, bufs, in_sems, out_sems):
    wid = lax.axis_index("s") * _NC + lax.axis_index("c")
    ii0 = lax.iota(jnp.int32, _L)
    lane0 = ii0 == 0

    pltpu.sync_copy(idx_hbm, idx_v.at[pl.ds(0, _NUM_UPD)])

    for b in range(_JW_SZ // _L):
        jwin[pl.ds(b * _L, _L)] = jnp.full((_L,), -1, jnp.int32)

    def sel_body(t, _):
        v = idx_v[pl.ds(t * _L, _L)]
        mine = (v & (_NW - 1)) == wid

        @pl.when(jnp.any(mine))
        def _():
            slots = lax.shift_right_logical(v, 5)
            jv = (t * _L) + ii0
            plsc.store_scatter(jwin, [slots], jv, mask=mine)
            g0 = plsc.load_gather(jwin, [slots], mask=mine)

            def fix_cond(g):
                return jnp.any(mine & (jv > g))

            def fix_body(g):
                plsc.store_scatter(jwin, [slots], jv, mask=mine & (jv > g))
                return plsc.load_gather(jwin, [slots], mask=mine)

            lax.while_loop(fix_cond, fix_body, g0)

        return 0

    lax.fori_loop(0, _NUM_UPD // _L, sel_body, 0)

    jlast = _sload(jwin, _SLOTS_W - 1)

    def pad_body(p, _):
        _sstore(jwin, p, jlast, lane0)
        return 0

    lax.fori_loop(_SLOTS_W, _ROWS_P, pad_body, 0)

    zero16 = jnp.zeros((_L,), jnp.float32)

    def slot_of(p):
        return jnp.minimum(p, _SLOTS_W - 1) * _NW + wid

    def fire_in(p, b):
        j = _sload(jwin, p)

        @pl.when(j >= 0)
        def _():
            pltpu.async_copy(arg3_hbm.at[j], bufs.at[b], in_sems.at[b])

        @pl.when(j < 0)
        def _():
            pltpu.async_copy(
                arg0_hbm.at[slot_of(p)], bufs.at[b], in_sems.at[b]
            )

    def drain_in(b):
        pltpu.make_async_copy(
            arg0_hbm.at[0], bufs.at[b], in_sems.at[b]
        ).wait()

    def fire_out(p, b):
        pltpu.async_copy(bufs.at[b], out_hbm.at[slot_of(p)], out_sems.at[b])

    def drain_out(b):
        pltpu.make_async_copy(
            bufs.at[b], out_hbm.at[0], out_sems.at[b]
        ).wait()

    def zero_diag(b):
        for k in range(_D // _L):
            ii = ii0 + (_L * k)
            plsc.store_scatter(bufs.at[b], [ii, ii], zero16)

    for b in range(_LOOK):
        fire_in(b, b)

    def ring_body(i, _):
        r0 = _R * i
        for t in range(_R):
            r = r0 + t
            drain_in(t)
            zero_diag(t)
            fire_out(r, t)
            to = (t + _LOOK) % _R
            if t >= _LAG:
                drain_out(to)
            else:

                @pl.when(r >= _LAG)
                def _():
                    drain_out(to)

            fire_in(r + _LOOK, to)
        return 0

    lax.fori_loop(0, _ROWS_P // _R, ring_body, 0)

    for t in range(_LAG):
        drain_out((_ROWS_P - _LAG + t) % _R)
    for t in range(_LOOK):
        drain_in((_ROWS_P + t) % _R)


@jax.jit
def kernel(arg0_1, arg1_1, arg2_1, arg3_1):
    del arg2_1
    idx = arg1_1.astype(jnp.int32)
    return _sc_update(idx, arg0_1, arg3_1)
